# baseline (speedup 1.0000x reference)
import functools
import math

import jax
import jax.numpy as jnp
from jax import lax
from jax.experimental import pallas as pl
from jax.experimental.pallas import tpu as pltpu
from jax.experimental.pallas import tpu_sc as plsc

_N = 4096
_SIZE = 32000
_PAD = 0
_EPS = 0.1 / (_SIZE - 2)
_CONF = 0.9
_C_ROW = (_SIZE - 2) * _EPS * math.log(_EPS) + _CONF * math.log(_CONF)


_BR = 1024
_BC = 3200
_GR = _N // _BR
_GC = _SIZE // _BC
_SUB = _BC // 128


def _s1_body(x_ref, trow_ref, rows_ref, chunks_ref):
    j = pl.program_id(1)

    xb = x_ref[...]
    partial = jnp.sum(xb, axis=1, keepdims=True)
    partial = partial - jnp.where(j == 0, xb[:, 0:1], 0.0)

    @pl.when(j == 0)
    def _rfirst():
        rows_ref[...] = partial

    @pl.when(j > 0)
    def _rrest():
        rows_ref[...] += partial

    tch = trow_ref[...] >> 7
    acc128 = jnp.zeros((_BR, 128), jnp.float32)
    for sc in range(_SUB):
        mf = (tch == (j * _SUB + sc)).astype(jnp.float32)
        acc128 = acc128 + xb[:, sc * 128 : (sc + 1) * 128] * mf

    @pl.when(j == 0)
    def _cfirst():
        chunks_ref[...] = acc128

    @pl.when(j > 0)
    def _crest():
        chunks_ref[...] += acc128


_s1_call = pl.pallas_call(
    _s1_body,
    grid=(_GR, _GC),
    in_specs=[
        pl.BlockSpec((_BR, _BC), lambda i, j: (i, j)),
        pl.BlockSpec((_BR, 1), lambda i, j: (i, 0)),
    ],
    out_specs=[
        pl.BlockSpec((_BR, 1), lambda i, j: (i, 0)),
        pl.BlockSpec((_BR, 128), lambda i, j: (i, 0)),
    ],
    out_shape=[
        jax.ShapeDtypeStruct((_N, 1), jnp.float32),
        jax.ShapeDtypeStruct((_N, 128), jnp.float32),
    ],
    compiler_params=pltpu.CompilerParams(
        dimension_semantics=("arbitrary", "arbitrary")
    ),
)


_NW = 32
_BPW = _N // _NW
_L = 16


def _sc_gather_body(cflat_hbm, tgt_hbm, out_hbm, tgt_v, idx_v, g_v, sem):
    c = lax.axis_index("c")
    s = lax.axis_index("s")
    wid = s * 2 + c
    base = wid * _BPW
    pltpu.sync_copy(tgt_hbm.at[pl.ds(base, _BPW)], tgt_v)
    for j in range(_BPW // _L):
        t = tgt_v[pl.ds(j * _L, _L)]
        rows = (base + j * _L) + lax.broadcasted_iota(jnp.int32, (_L,), 0)
        idx_v[pl.ds(j * _L, _L)] = rows * 128 + (t & 127)
    pltpu.async_copy(cflat_hbm.at[idx_v], g_v, sem).wait()
    pltpu.sync_copy(g_v, out_hbm.at[pl.ds(base, _BPW)])


@functools.cache
def _sc_gather():
    return functools.partial(
        pl.kernel,
        mesh=plsc.VectorSubcoreMesh(core_axis_name="c", subcore_axis_name="s"),
        out_type=jax.ShapeDtypeStruct((_N,), jnp.float32),
        scratch_types=[
            pltpu.VMEM((_BPW,), jnp.int32),
            pltpu.VMEM((_BPW,), jnp.int32),
            pltpu.VMEM((_BPW,), jnp.float32),
            pltpu.SemaphoreType.DMA,
        ],
    )(_sc_gather_body)


def _combine_body(r2d_ref, t2d_ref, g2d_ref, out_ref):
    m = t2d_ref[...] != _PAD
    k = jnp.sum(jnp.where(m, 1.0, 0.0))
    s1 = jnp.sum(jnp.where(m, r2d_ref[...], 0.0))
    gsum = jnp.sum(jnp.where(m, g2d_ref[...], 0.0))
    out_ref[0, 0] = (k * _C_ROW - _EPS * s1 - (_CONF - _EPS) * gsum) * (
        1.0 / _N
    )


_combine_call = pl.pallas_call(
    _combine_body,
    in_specs=[
        pl.BlockSpec((_N // 128, 128), lambda: (0, 0)),
        pl.BlockSpec((_N // 128, 128), lambda: (0, 0)),
        pl.BlockSpec((_N // 128, 128), lambda: (0, 0)),
    ],
    out_specs=pl.BlockSpec(memory_space=pltpu.SMEM),
    out_shape=jax.ShapeDtypeStruct((1, 1), jnp.float32),
)


def kernel(x, target):
    rows, chunks = _s1_call(x, target.reshape(_N, 1))
    g = _sc_gather()(chunks.reshape(-1), target)
    loss = _combine_call(
        rows.reshape(_N // 128, 128),
        target.reshape(_N // 128, 128),
        g.reshape(_N // 128, 128),
    )
    return loss[0, 0]


# speedup vs baseline: 1.0279x; 1.0279x over previous
import functools
import math

import jax
import jax.numpy as jnp
from jax import lax
from jax.experimental import pallas as pl
from jax.experimental.pallas import tpu as pltpu
from jax.experimental.pallas import tpu_sc as plsc

_N = 4096
_SIZE = 32000
_PAD = 0
_EPS = 0.1 / (_SIZE - 2)
_CONF = 0.9
_C_ROW = (_SIZE - 2) * _EPS * math.log(_EPS) + _CONF * math.log(_CONF)


_BR = 1024
_BC = 3200
_GR = _N // _BR
_GC = _SIZE // _BC
_SUB = _BC // 128


def _s1_body(x_ref, trow_ref, rows_ref, chunks_ref):
    j = pl.program_id(1)

    xb = x_ref[...]
    tch = trow_ref[...] >> 7
    lane2d = lax.broadcasted_iota(jnp.int32, (_BR, 128), 1)
    partial = -jnp.where(
        (lane2d == 0) & (j == 0), xb[:, 0:1], 0.0
    )
    acc128 = jnp.zeros((_BR, 128), jnp.float32)
    for sc in range(_SUB):
        slab = xb[:, sc * 128 : (sc + 1) * 128]
        partial = partial + slab
        mf = (tch == (j * _SUB + sc)).astype(jnp.float32)
        acc128 = acc128 + slab * mf

    @pl.when(j == 0)
    def _rfirst():
        rows_ref[...] = partial

    @pl.when(j > 0)
    def _rrest():
        rows_ref[...] += partial

    @pl.when(j == 0)
    def _cfirst():
        chunks_ref[...] = acc128

    @pl.when(j > 0)
    def _crest():
        chunks_ref[...] += acc128


_s1_call = pl.pallas_call(
    _s1_body,
    grid=(_GR, _GC),
    in_specs=[
        pl.BlockSpec((_BR, _BC), lambda i, j: (i, j)),
        pl.BlockSpec((_BR, 1), lambda i, j: (i, 0)),
    ],
    out_specs=[
        pl.BlockSpec((_BR, 128), lambda i, j: (i, 0)),
        pl.BlockSpec((_BR, 128), lambda i, j: (i, 0)),
    ],
    out_shape=[
        jax.ShapeDtypeStruct((_N, 128), jnp.float32),
        jax.ShapeDtypeStruct((_N, 128), jnp.float32),
    ],
    compiler_params=pltpu.CompilerParams(
        dimension_semantics=("arbitrary", "arbitrary")
    ),
)


_NW = 32
_BPW = _N // _NW
_L = 16


def _sc_gather_body(cflat_hbm, tgt_hbm, out_hbm, tgt_v, idx_v, g_v, sem):
    c = lax.axis_index("c")
    s = lax.axis_index("s")
    wid = s * 2 + c
    base = wid * _BPW
    pltpu.sync_copy(tgt_hbm.at[pl.ds(base, _BPW)], tgt_v)
    for j in range(_BPW // _L):
        t = tgt_v[pl.ds(j * _L, _L)]
        rows = (base + j * _L) + lax.broadcasted_iota(jnp.int32, (_L,), 0)
        idx_v[pl.ds(j * _L, _L)] = rows * 128 + (t & 127)
    pltpu.async_copy(cflat_hbm.at[idx_v], g_v, sem).wait()
    pltpu.sync_copy(g_v, out_hbm.at[pl.ds(base, _BPW)])


@functools.cache
def _sc_gather():
    return functools.partial(
        pl.kernel,
        mesh=plsc.VectorSubcoreMesh(core_axis_name="c", subcore_axis_name="s"),
        out_type=jax.ShapeDtypeStruct((_N,), jnp.float32),
        scratch_types=[
            pltpu.VMEM((_BPW,), jnp.int32),
            pltpu.VMEM((_BPW,), jnp.int32),
            pltpu.VMEM((_BPW,), jnp.float32),
            pltpu.SemaphoreType.DMA,
        ],
    )(_sc_gather_body)


def _combine_body(rl_ref, tcol_ref, t2d_ref, g2d_ref, out_ref):
    m = t2d_ref[...] != _PAD
    k = jnp.sum(jnp.where(m, 1.0, 0.0))
    mrow = tcol_ref[...] != _PAD
    s1 = jnp.sum(jnp.where(mrow, rl_ref[...], 0.0))
    gsum = jnp.sum(jnp.where(m, g2d_ref[...], 0.0))
    out_ref[0, 0] = (k * _C_ROW - _EPS * s1 - (_CONF - _EPS) * gsum) * (
        1.0 / _N
    )


_combine_call = pl.pallas_call(
    _combine_body,
    in_specs=[
        pl.BlockSpec((_N, 128), lambda: (0, 0)),
        pl.BlockSpec((_N, 1), lambda: (0, 0)),
        pl.BlockSpec((_N // 128, 128), lambda: (0, 0)),
        pl.BlockSpec((_N // 128, 128), lambda: (0, 0)),
    ],
    out_specs=pl.BlockSpec(memory_space=pltpu.SMEM),
    out_shape=jax.ShapeDtypeStruct((1, 1), jnp.float32),
)


def kernel(x, target):
    rows, chunks = _s1_call(x, target.reshape(_N, 1))
    g = _sc_gather()(chunks.reshape(-1), target)
    loss = _combine_call(
        rows,
        target.reshape(_N, 1),
        target.reshape(_N // 128, 128),
        g.reshape(_N // 128, 128),
    )
    return loss[0, 0]


# speedup vs baseline: 1.0698x; 1.0407x over previous
import functools
import math

import jax
import jax.numpy as jnp
from jax import lax
from jax.experimental import pallas as pl
from jax.experimental.pallas import tpu as pltpu
from jax.experimental.pallas import tpu_sc as plsc

_N = 4096
_SIZE = 32000
_PAD = 0
_EPS = 0.1 / (_SIZE - 2)
_CONF = 0.9
_C_ROW = (_SIZE - 2) * _EPS * math.log(_EPS) + _CONF * math.log(_CONF)


_BR = 1024
_BC = 6400
_GR = _N // _BR
_GC = _SIZE // _BC
_SUB = _BC // 128


def _s1_body(x_ref, trow_ref, rows_ref, chunks_ref):
    j = pl.program_id(1)

    xb = x_ref[...]
    tch = trow_ref[...] >> 7
    lane2d = lax.broadcasted_iota(jnp.int32, (_BR, 128), 1)
    partial = -jnp.where(
        (lane2d == 0) & (j == 0), xb[:, 0:1], 0.0
    )
    acc128 = jnp.zeros((_BR, 128), jnp.float32)
    for sc in range(_SUB):
        slab = xb[:, sc * 128 : (sc + 1) * 128]
        partial = partial + slab
        mf = (tch == (j * _SUB + sc)).astype(jnp.float32)
        acc128 = acc128 + slab * mf

    @pl.when(j == 0)
    def _rfirst():
        rows_ref[...] = partial

    @pl.when(j > 0)
    def _rrest():
        rows_ref[...] += partial

    @pl.when(j == 0)
    def _cfirst():
        chunks_ref[...] = acc128

    @pl.when(j > 0)
    def _crest():
        chunks_ref[...] += acc128


_s1_call = pl.pallas_call(
    _s1_body,
    grid=(_GR, _GC),
    in_specs=[
        pl.BlockSpec((_BR, _BC), lambda i, j: (i, j)),
        pl.BlockSpec((_BR, 1), lambda i, j: (i, 0)),
    ],
    out_specs=[
        pl.BlockSpec((_BR, 128), lambda i, j: (i, 0)),
        pl.BlockSpec((_BR, 128), lambda i, j: (i, 0)),
    ],
    out_shape=[
        jax.ShapeDtypeStruct((_N, 128), jnp.float32),
        jax.ShapeDtypeStruct((_N, 128), jnp.float32),
    ],
    compiler_params=pltpu.CompilerParams(
        dimension_semantics=("arbitrary", "arbitrary")
    ),
)


_NW = 32
_BPW = _N // _NW
_L = 16


def _sc_gather_body(cflat_hbm, tgt_hbm, out_hbm, tgt_v, idx_v, g_v, sem):
    c = lax.axis_index("c")
    s = lax.axis_index("s")
    wid = s * 2 + c
    base = wid * _BPW
    pltpu.sync_copy(tgt_hbm.at[pl.ds(base, _BPW)], tgt_v)
    for j in range(_BPW // _L):
        t = tgt_v[pl.ds(j * _L, _L)]
        rows = (base + j * _L) + lax.broadcasted_iota(jnp.int32, (_L,), 0)
        idx_v[pl.ds(j * _L, _L)] = rows * 128 + (t & 127)
    pltpu.async_copy(cflat_hbm.at[idx_v], g_v, sem).wait()
    pltpu.sync_copy(g_v, out_hbm.at[pl.ds(base, _BPW)])


@functools.cache
def _sc_gather():
    return functools.partial(
        pl.kernel,
        mesh=plsc.VectorSubcoreMesh(core_axis_name="c", subcore_axis_name="s"),
        out_type=jax.ShapeDtypeStruct((_N,), jnp.float32),
        scratch_types=[
            pltpu.VMEM((_BPW,), jnp.int32),
            pltpu.VMEM((_BPW,), jnp.int32),
            pltpu.VMEM((_BPW,), jnp.float32),
            pltpu.SemaphoreType.DMA,
        ],
    )(_sc_gather_body)


def _combine_body(rl_ref, tcol_ref, t2d_ref, g2d_ref, out_ref):
    m = t2d_ref[...] != _PAD
    k = jnp.sum(jnp.where(m, 1.0, 0.0))
    mrow = tcol_ref[...] != _PAD
    s1 = jnp.sum(jnp.where(mrow, rl_ref[...], 0.0))
    gsum = jnp.sum(jnp.where(m, g2d_ref[...], 0.0))
    out_ref[0, 0] = (k * _C_ROW - _EPS * s1 - (_CONF - _EPS) * gsum) * (
        1.0 / _N
    )


_combine_call = pl.pallas_call(
    _combine_body,
    in_specs=[
        pl.BlockSpec((_N, 128), lambda: (0, 0)),
        pl.BlockSpec((_N, 1), lambda: (0, 0)),
        pl.BlockSpec((_N // 128, 128), lambda: (0, 0)),
        pl.BlockSpec((_N // 128, 128), lambda: (0, 0)),
    ],
    out_specs=pl.BlockSpec(memory_space=pltpu.SMEM),
    out_shape=jax.ShapeDtypeStruct((1, 1), jnp.float32),
)


def kernel(x, target):
    rows, chunks = _s1_call(x, target.reshape(_N, 1))
    g = _sc_gather()(chunks.reshape(-1), target)
    loss = _combine_call(
        rows,
        target.reshape(_N, 1),
        target.reshape(_N // 128, 128),
        g.reshape(_N // 128, 128),
    )
    return loss[0, 0]
